# baseline (speedup 1.0000x reference)
import functools

import jax
import jax.numpy as jnp
from jax import lax
from jax.experimental import pallas as pl
from jax.experimental.pallas import tpu as pltpu
from jax.experimental.pallas import tpu_sc as plsc
from jax.experimental.layout import Layout, Format, with_layout_constraint


def kernel(x, table):
    B, H = x.shape
    V, D = table.shape
    NBT = B // 128

    t_lin = table
    xt = x.T

    info = plsc.get_sparse_core_info()
    NC = info.num_cores
    NW = NC * info.num_subcores
    n_pairs = H * NBT
    per_w = n_pairs // NW

    mesh = plsc.VectorSubcoreMesh(core_axis_name="c", subcore_axis_name="s")

    @functools.partial(
        pl.kernel,
        mesh=mesh,
        out_type=jax.ShapeDtypeStruct((H, 8, NBT, 8, 128), jnp.float32),
        compiler_params=pltpu.CompilerParams(
            use_tc_tiling_on_sc=False, needs_layout_passes=False
        ),
        scratch_types=[
            pltpu.VMEM((128,), jnp.int32),
            pltpu.VMEM((128, D), jnp.float32),
            pltpu.VMEM((8, 8, 128), jnp.float32),
            pltpu.SemaphoreType.DMA,
        ],
    )
    def emb_kernel(xt_hbm, table_hbm, out_hbm, idx_v, rows_v, trans_v, sem):
        wid = lax.axis_index("s") * NC + lax.axis_index("c")
        iota = lax.iota(jnp.int32, 16)

        def body(p, carry):
            pg = wid * per_w + p
            h = pg // NBT
            bt = pg % NBT
            pltpu.sync_copy(xt_hbm.at[h, pl.ds(bt * 128, 128)], idx_v)
            pltpu.async_copy(table_hbm.at[idx_v], rows_v, sem).wait()
            for fg in range(8):
                for f in range(8):
                    col = jnp.full((16,), fg * 8 + f, jnp.int32)
                    for tg in range(8):
                        vec = plsc.load_gather(
                            rows_v, [tg * 16 + iota, col]
                        )
                        trans_v[fg, f, pl.ds(tg * 16, 16)] = vec
            pltpu.sync_copy(trans_v, out_hbm.at[h, :, bt])
            return carry

        lax.fori_loop(0, per_w, body, None)

    out5 = emb_kernel(xt, t_lin)
    return out5.transpose(2, 4, 0, 1, 3).reshape(B, H, D)


# speedup vs baseline: 1.3082x; 1.3082x over previous
import functools

import jax
import jax.numpy as jnp
from jax import lax
from jax.experimental import pallas as pl
from jax.experimental.pallas import tpu as pltpu
from jax.experimental.pallas import tpu_sc as plsc
from jax.experimental.layout import Layout, with_layout_constraint


def kernel(x, table):
    B, H = x.shape
    V, D = table.shape
    NBT = B // 128

    t_lin = table
    xt = x.T

    info = plsc.get_sparse_core_info()
    NC = info.num_cores
    NW = NC * info.num_subcores
    n_pairs = H * NBT
    per_w = n_pairs // NW

    mesh = plsc.VectorSubcoreMesh(core_axis_name="c", subcore_axis_name="s")

    @functools.partial(
        pl.kernel,
        mesh=mesh,
        out_type=jax.ShapeDtypeStruct((H, 8, NBT, 8, 128), jnp.float32),
        compiler_params=pltpu.CompilerParams(
            use_tc_tiling_on_sc=False, needs_layout_passes=False
        ),
        scratch_types=[
            pltpu.VMEM((2, 128), jnp.int32),
            pltpu.VMEM((256, D), jnp.float32),
            pltpu.VMEM((2, 8, 8, 128), jnp.float32),
            pltpu.SemaphoreType.DMA,
            pltpu.SemaphoreType.DMA,
            pltpu.SemaphoreType.DMA,
            pltpu.SemaphoreType.DMA,
        ],
    )
    def emb_kernel(xt_hbm, table_hbm, out_hbm, idx_v, rows_v, trans_v,
                   g0, g1, w0, w1):
        sem_g = (g0, g1)
        sem_w = (w0, w1)
        wid = lax.axis_index("s") * NC + lax.axis_index("c")
        iota = lax.iota(jnp.int32, 16)
        base = wid * per_w

        def load_and_fire(p, b):
            pltpu.sync_copy(
                xt_hbm.at[(base + p) // NBT,
                          pl.ds(((base + p) % NBT) * 128, 128)],
                idx_v.at[b],
            )
            pltpu.async_copy(
                table_hbm.at[idx_v.at[b]],
                rows_v.at[pl.ds(b * 128, 128)],
                sem_g[b],
            )

        def wait_gather(b):
            pltpu.make_async_copy(
                table_hbm.at[idx_v.at[b]],
                rows_v.at[pl.ds(b * 128, 128)],
                sem_g[b],
            ).wait()

        def transpose(b):
            for fg in range(8):
                for f in range(8):
                    col = jnp.full((16,), fg * 8 + f, jnp.int32)
                    vecs = [
                        plsc.load_gather(
                            rows_v, [b * 128 + tg * 16 + iota, col]
                        )
                        for tg in range(8)
                    ]
                    for tg in range(8):
                        trans_v[b, fg, f, pl.ds(tg * 16, 16)] = vecs[tg]

        def start_write(p, b):
            pltpu.async_copy(
                trans_v.at[b],
                out_hbm.at[(base + p) // NBT, :, (base + p) % NBT],
                sem_w[b],
            )

        def wait_write(b):
            pltpu.make_async_copy(
                trans_v.at[b], out_hbm.at[0, :, 0], sem_w[b]
            ).wait()

        load_and_fire(0, 0)

        def body(q, carry):
            for b in range(2):
                p = q * 2 + b

                @pl.when(p < per_w - 1)
                def _():
                    load_and_fire(p + 1, 1 - b)

                @pl.when(p >= 2)
                def _():
                    wait_write(b)

                wait_gather(b)
                transpose(b)
                start_write(p, b)
            return carry

        lax.fori_loop(0, per_w // 2, body, None)
        wait_write(0)
        wait_write(1)

    out5 = emb_kernel(xt, t_lin)
    return out5.transpose(2, 4, 0, 1, 3).reshape(B, H, D)


# speedup vs baseline: 1.3750x; 1.0511x over previous
import functools

import jax
import jax.numpy as jnp
from jax import lax
from jax.experimental import pallas as pl
from jax.experimental.pallas import tpu as pltpu
from jax.experimental.pallas import tpu_sc as plsc


def kernel(x, table):
    B, H = x.shape
    V, D = table.shape
    NBT = B // 128

    t_lin = table
    xt2 = x.T.reshape(H * NBT, 128)

    info = plsc.get_sparse_core_info()
    NC = info.num_cores
    NW = NC * info.num_subcores
    n_pairs = H * NBT
    per_w = n_pairs // NW
    NBUF = 2

    mesh = plsc.VectorSubcoreMesh(core_axis_name="c", subcore_axis_name="s")

    @functools.partial(
        pl.kernel,
        mesh=mesh,
        out_type=jax.ShapeDtypeStruct((H, 8, NBT, 8, 128), jnp.float32),
        compiler_params=pltpu.CompilerParams(
            use_tc_tiling_on_sc=False, needs_layout_passes=False
        ),
        scratch_types=[
            pltpu.VMEM((per_w, 128), jnp.int32),
            pltpu.VMEM((NBUF * 128, D), jnp.float32),
            pltpu.VMEM((2, 8, 8, 128), jnp.float32),
        ] + [pltpu.SemaphoreType.DMA] * (NBUF + 2),
    )
    def emb_kernel(xt_hbm, table_hbm, out_hbm, idx_v, rows_v, trans_v, *sems):
        sem_g = sems[:NBUF]
        sem_w = sems[NBUF:]
        wid = lax.axis_index("s") * NC + lax.axis_index("c")
        iota = lax.iota(jnp.int32, 16)
        base = wid * per_w

        pltpu.sync_copy(xt_hbm.at[pl.ds(base, per_w)], idx_v)

        def fire_gather(p, g):
            pltpu.async_copy(
                table_hbm.at[idx_v.at[p]],
                rows_v.at[pl.ds(g * 128, 128)],
                sem_g[g],
            )

        def wait_gather(g):
            pltpu.make_async_copy(
                table_hbm.at[idx_v.at[0]],
                rows_v.at[pl.ds(g * 128, 128)],
                sem_g[g],
            ).wait()

        def transpose(g, t):
            for fg in range(8):
                for f in range(8):
                    col = jnp.full((16,), fg * 8 + f, jnp.int32)
                    vecs = [
                        plsc.load_gather(
                            rows_v, [g * 128 + tg * 16 + iota, col]
                        )
                        for tg in range(8)
                    ]
                    for tg in range(8):
                        trans_v[t, fg, f, pl.ds(tg * 16, 16)] = vecs[tg]

        def start_write(p, t):
            pltpu.async_copy(
                trans_v.at[t],
                out_hbm.at[(base + p) // NBT, :, (base + p) % NBT],
                sem_w[t],
            )

        def wait_write(t):
            pltpu.make_async_copy(
                trans_v.at[t], out_hbm.at[0, :, 0], sem_w[t]
            ).wait()

        for g in range(NBUF):
            fire_gather(g, g)

        def body(q, carry):
            for r in range(2):
                p = q * 2 + r
                g = r
                t = r
                wait_gather(g)

                @pl.when(p >= 2)
                def _():
                    wait_write(t)

                transpose(g, t)

                @pl.when(p + NBUF < per_w)
                def _():
                    fire_gather(p + NBUF, g)

                start_write(p, t)
            return carry

        lax.fori_loop(0, per_w // 2, body, None)
        wait_write(0)
        wait_write(1)

    out5 = emb_kernel(xt2, t_lin)
    return out5.transpose(2, 4, 0, 1, 3).reshape(B, H, D)


# speedup vs baseline: 2.1335x; 1.5516x over previous
import functools

import jax
import jax.numpy as jnp
from jax import lax
from jax.experimental import pallas as pl
from jax.experimental.pallas import tpu as pltpu
from jax.experimental.pallas import tpu_sc as plsc


def kernel(x, table):
    B, H = x.shape
    V, D = table.shape
    NBT = B // 128

    t_lin = table
    xt2 = x.T.reshape(H * NBT, 128)

    info = plsc.get_sparse_core_info()
    NC = info.num_cores
    NW = NC * info.num_subcores
    n_pairs = H * NBT
    per_w = n_pairs // NW
    NBUF = 2

    mesh = plsc.VectorSubcoreMesh(core_axis_name="c", subcore_axis_name="s")

    @functools.partial(
        pl.kernel,
        mesh=mesh,
        out_type=jax.ShapeDtypeStruct((H, 8, NBT, 8, 128), jnp.float32),
        compiler_params=pltpu.CompilerParams(
            use_tc_tiling_on_sc=False, needs_layout_passes=False
        ),
        scratch_types=[
            pltpu.VMEM((per_w, 128), jnp.int32),
            pltpu.VMEM((NBUF * 128, D), jnp.float32),
            pltpu.VMEM((2, 8, 8, 128), jnp.float32),
        ] + [pltpu.SemaphoreType.DMA] * (NBUF + 2),
    )
    def emb_kernel(xt_hbm, table_hbm, out_hbm, idx_v, rows_v, trans_v, *sems):
        sem_g = sems[:NBUF]
        sem_w = sems[NBUF:]
        wid = lax.axis_index("s") * NC + lax.axis_index("c")
        iota = lax.iota(jnp.int32, 16)
        base = wid * per_w

        pltpu.sync_copy(xt_hbm.at[pl.ds(base, per_w)], idx_v)

        def fire_gather(p, g):
            pltpu.async_copy(
                table_hbm.at[idx_v.at[p]],
                rows_v.at[pl.ds(g * 128, 128)],
                sem_g[g],
            )

        def wait_gather(g):
            pltpu.make_async_copy(
                table_hbm.at[idx_v.at[0]],
                rows_v.at[pl.ds(g * 128, 128)],
                sem_g[g],
            ).wait()

        def transpose(g, t):
            def fg_body(fg, carry):
                for f in range(8):
                    c = fg * 8 + f
                    col = lax.rem(iota + c, 64)
                    fgv = lax.shift_right_logical(col, 3)
                    fv = lax.bitwise_and(col, 7)
                    for tg in range(8):
                        vec = plsc.load_gather(
                            rows_v, [g * 128 + tg * 16 + iota, col]
                        )
                        plsc.store_scatter(
                            trans_v.at[t], [fgv, fv, tg * 16 + iota], vec
                        )
                return carry

            lax.fori_loop(0, 8, fg_body, None)

        def start_write(p, t):
            pltpu.async_copy(
                trans_v.at[t],
                out_hbm.at[(base + p) // NBT, :, (base + p) % NBT],
                sem_w[t],
            )

        def wait_write(t):
            pltpu.make_async_copy(
                trans_v.at[t], out_hbm.at[0, :, 0], sem_w[t]
            ).wait()

        for g in range(NBUF):
            fire_gather(g, g)

        def body(q, carry):
            for r in range(2):
                p = q * 2 + r
                g = r
                t = r
                wait_gather(g)

                @pl.when(p >= 2)
                def _():
                    wait_write(t)

                transpose(g, t)

                @pl.when(p + NBUF < per_w)
                def _():
                    fire_gather(p + NBUF, g)

                start_write(p, t)
            return carry

        lax.fori_loop(0, per_w // 2, body, None)
        wait_write(0)
        wait_write(1)

    out5 = emb_kernel(xt2, t_lin)
    return out5.transpose(2, 4, 0, 1, 3).reshape(B, H, D)
